# Initial kernel scaffold; baseline (speedup 1.0000x reference)
import jax
import jax.numpy as jnp
from jax.experimental import pallas as pl


def kernel(x, contig_ei, alliance_ei, trade_ei, W_c, b_c, W_a, b_a, W_t, b_t, W2, b2, ln1_w, ln1_b, ln2_w, ln2_b):
    raise NotImplementedError("write your pallas kernel here")


# speedup vs baseline: 12.8352x; 12.8352x over previous
import functools

import jax
import jax.numpy as jnp
from jax import lax
from jax.experimental import pallas as pl
from jax.experimental.pallas import tpu as pltpu
from jax.experimental.pallas import tpu_sc as plsc

_N = 10000
_E = 320000
_D = 128
_H = 128

_NC = 2
_NS = 16
_NW = _NC * _NS
_ET = _E // _NW
_CH = 128
_NFULL = _ET // _CH
_TAIL = _ET - _NFULL * _CH
_NPAD = 10240
_RPT = _NPAD // _NS
_ZR = 64
_BM = 1000


def _mesh():
  return plsc.VectorSubcoreMesh(
      core_axis_name="c", subcore_axis_name="s",
      num_cores=_NC, num_subcores=_NS)


def _make_agg(rels):
  scratch = [
      pltpu.VMEM((_CH,), jnp.int32),
      pltpu.VMEM((_CH,), jnp.int32),
      pltpu.VMEM((_TAIL,), jnp.int32),
      pltpu.VMEM((_TAIL,), jnp.int32),
      pltpu.VMEM((_CH, _H), jnp.float32),
      pltpu.VMEM((_TAIL, _H), jnp.float32),
      pltpu.VMEM((_CH, _H), jnp.float32),
      pltpu.VMEM((_CH,), jnp.int32),
      pltpu.VMEM((_CH,), jnp.int32),
      pltpu.VMEM_SHARED((_NPAD, _H), jnp.float32),
      pltpu.SemaphoreType.DMA,
  ]

  @functools.partial(
      pl.kernel,
      out_type=[jax.ShapeDtypeStruct((_NC, _NPAD, _H), jnp.float32)] * rels,
      mesh=_mesh(),
      scratch_types=scratch,
  )
  def agg(*refs):
    gs = refs[:rels]
    srcs = refs[rels:2 * rels]
    dsts = refs[2 * rels:3 * rels]
    outs = refs[3 * rels:4 * rels]
    sv, dv, svt, dvt, rows, rowst, zrows, riota, zidx, acc, sem = refs[4 * rels:]
    cid = lax.axis_index("c")
    sid = lax.axis_index("s")
    ebase = cid * (_E // _NC) + sid * _ET

    def zfill(i, _):
      for u in range(_H // 16):
        zrows[i, pl.ds(u * 16, 16)] = jnp.zeros((16,), jnp.float32)
      return 0
    lax.fori_loop(0, _CH, zfill, 0)
    for u in range(_CH // 16):
      riota[pl.ds(u * 16, 16)] = (
          lax.broadcasted_iota(jnp.int32, (16,), 0) + u * 16)

    for g, esrc, edst, out in zip(gs, srcs, dsts, outs):
      for k in range(_RPT // _CH):
        for u in range(_CH // 16):
          zidx[pl.ds(u * 16, 16)] = (
              riota[pl.ds(u * 16, 16)] + (sid * _RPT + k * _CH))
        pltpu.sync_copy(zrows, acc.at[zidx])
      plsc.subcore_barrier()

      def body(j, _):
        pltpu.sync_copy(esrc.at[pl.ds(ebase + j * _CH, _CH)], sv)
        pltpu.sync_copy(edst.at[pl.ds(ebase + j * _CH, _CH)], dv)
        pltpu.async_copy(g.at[sv], rows, sem).wait()
        pltpu.sync_copy(rows, acc.at[dv], add=True)
        return 0
      lax.fori_loop(0, _NFULL, body, 0)

      pltpu.sync_copy(esrc.at[pl.ds(ebase + _NFULL * _CH, _TAIL)], svt)
      pltpu.sync_copy(edst.at[pl.ds(ebase + _NFULL * _CH, _TAIL)], dvt)
      pltpu.async_copy(g.at[svt], rowst, sem).wait()
      pltpu.sync_copy(rowst, acc.at[dvt], add=True)
      plsc.subcore_barrier()
      for k in range(_RPT // _CH):
        for u in range(_CH // 16):
          zidx[pl.ds(u * 16, 16)] = (
              riota[pl.ds(u * 16, 16)] + (sid * _RPT + k * _CH))
        pltpu.async_copy(acc.at[zidx], rows, sem).wait()
        pltpu.sync_copy(rows,
                        out.at[cid, pl.ds(sid * _RPT + k * _CH, _CH), :])
      plsc.subcore_barrier()

  return agg


_agg3 = _make_agg(3)
_agg1 = _make_agg(1)


def _deg_inv(hist_block):
  deg = (jnp.sum(hist_block[0], axis=-1, keepdims=True)
         + jnp.sum(hist_block[1], axis=-1, keepdims=True)) * (1.0 / _H) + 1.0
  return lax.rsqrt(deg)


def _tc_stage1(x, Wc, Wa, Wt, hc, ha, ht):
  grid = (_N // _BM,)
  bx = pl.BlockSpec((_BM, _D), lambda i: (i, 0))
  bw = pl.BlockSpec((_D, _H), lambda i: (0, 0))
  bh = pl.BlockSpec((_NC, _BM, _H), lambda i: (0, i, 0))
  bo = pl.BlockSpec((_BM, _H), lambda i: (i, 0))

  def body(x_r, wc_r, wa_r, wt_r, hc_r, ha_r, ht_r, gc_r, ga_r, gt_r):
    xb = x_r[...]
    for w_r, h_r, g_r in ((wc_r, hc_r, gc_r), (wa_r, ha_r, ga_r),
                          (wt_r, ht_r, gt_r)):
      dinv = _deg_inv(h_r[...])
      g_r[...] = dinv * jnp.dot(xb, w_r[...],
                                preferred_element_type=jnp.float32)

  return pl.pallas_call(
      body, grid=grid,
      in_specs=[bx, bw, bw, bw, bh, bh, bh],
      out_specs=[bo, bo, bo],
      out_shape=[jax.ShapeDtypeStruct((_N, _H), jnp.float32)] * 3,
  )(x, Wc, Wa, Wt, hc, ha, ht)


def _tc_stage2(pc, pa, pt, gc, ga, gt, hc, ha, ht, bsum, ln1w, ln1b, W2):
  grid = (_N // _BM,)
  bp = pl.BlockSpec((_NC, _BM, _H), lambda i: (0, i, 0))
  bg = pl.BlockSpec((_BM, _H), lambda i: (i, 0))
  bh = pl.BlockSpec((_NC, _BM, _H), lambda i: (0, i, 0))
  bv = pl.BlockSpec((1, _H), lambda i: (0, 0))
  bw = pl.BlockSpec((_H, _H), lambda i: (0, 0))

  def body(pc_r, pa_r, pt_r, gc_r, ga_r, gt_r, hc_r, ha_r, ht_r, bsum_r,
           w1_r, b1_r, w2m_r, g2_r):
    s = jnp.zeros((_BM, _H), jnp.float32) + bsum_r[...]
    dinv_c = None
    for p_r, g_r, h_r in ((pc_r, gc_r, hc_r), (pa_r, ga_r, ha_r),
                          (pt_r, gt_r, ht_r)):
      dinv = _deg_inv(h_r[...])
      if dinv_c is None:
        dinv_c = dinv
      pp = p_r[...]
      s = s + dinv * (pp[0] + pp[1] + g_r[...])
    mu = jnp.mean(s, axis=-1, keepdims=True)
    xc = s - mu
    var = jnp.mean(xc * xc, axis=-1, keepdims=True)
    hn = xc * lax.rsqrt(var + 1e-5) * w1_r[...] + b1_r[...]
    hh = jnp.where(hn > 0, hn, (jnp.exp(hn) - 1.0))
    g2_r[...] = dinv_c * jnp.dot(hh, w2m_r[...],
                                 preferred_element_type=jnp.float32)

  return pl.pallas_call(
      body, grid=grid,
      in_specs=[bp, bp, bp, bg, bg, bg, bh, bh, bh, bv, bv, bv, bw],
      out_specs=bg,
      out_shape=jax.ShapeDtypeStruct((_N, _H), jnp.float32),
  )(pc, pa, pt, gc, ga, gt, hc, ha, ht, bsum, ln1w, ln1b, W2)


def _tc_stage3(p2, g2, hc, b2, ln2w, ln2b):
  grid = (_N // _BM,)
  bp = pl.BlockSpec((_NC, _BM, _H), lambda i: (0, i, 0))
  bg = pl.BlockSpec((_BM, _H), lambda i: (i, 0))
  bh = pl.BlockSpec((_NC, _BM, _H), lambda i: (0, i, 0))
  bv = pl.BlockSpec((1, _H), lambda i: (0, 0))

  def body(p2_r, g2_r, hc_r, b2_r, w2_r, bb2_r, out_r):
    dinv = _deg_inv(hc_r[...])
    pp = p2_r[...]
    s = dinv * (pp[0] + pp[1] + g2_r[...]) + b2_r[...]
    mu = jnp.mean(s, axis=-1, keepdims=True)
    xc = s - mu
    var = jnp.mean(xc * xc, axis=-1, keepdims=True)
    hn = xc * lax.rsqrt(var + 1e-5) * w2_r[...] + bb2_r[...]
    out_r[...] = jnp.where(hn > 0, hn, (jnp.exp(hn) - 1.0))

  return pl.pallas_call(
      body, grid=grid,
      in_specs=[bp, bg, bh, bv, bv, bv],
      out_specs=bg,
      out_shape=jax.ShapeDtypeStruct((_N, _H), jnp.float32),
  )(p2, g2, hc, b2, ln2w, ln2b)


def _impl(x, contig_ei, alliance_ei, trade_ei, W_c, b_c, W_a, b_a, W_t, b_t,
          W2, b2, ln1_w, ln1_b, ln2_w, ln2_b):
  cs, cd = contig_ei[0].astype(jnp.int32), contig_ei[1].astype(jnp.int32)
  as_, ad = alliance_ei[0].astype(jnp.int32), alliance_ei[1].astype(jnp.int32)
  ts, td = trade_ei[0].astype(jnp.int32), trade_ei[1].astype(jnp.int32)

  ones_mat = jnp.ones((_N, _H), jnp.float32)
  hc, ha, ht = _agg3(ones_mat, ones_mat, ones_mat, cd, ad, td, cd, ad, td)
  gc, ga, gt = _tc_stage1(x, W_c, W_a, W_t, hc, ha, ht)
  pc, pa, pt = _agg3(gc, ga, gt, cs, as_, ts, cd, ad, td)
  bsum = (b_c + b_a + b_t).reshape(1, _H)
  g2 = _tc_stage2(pc, pa, pt, gc, ga, gt, hc, ha, ht, bsum,
                  ln1_w.reshape(1, _H), ln1_b.reshape(1, _H), W2)
  (p2,) = _agg1(g2, cs, cd)
  return _tc_stage3(p2, g2, hc, b2.reshape(1, _H),
                    ln2_w.reshape(1, _H), ln2_b.reshape(1, _H))


def kernel(x, contig_ei, alliance_ei, trade_ei, W_c, b_c, W_a, b_a, W_t, b_t,
           W2, b2, ln1_w, ln1_b, ln2_w, ln2_b):
  return _impl(x, contig_ei, alliance_ei, trade_ei, W_c, b_c, W_a, b_a,
               W_t, b_t, W2, b2, ln1_w, ln1_b, ln2_w, ln2_b)
